# baseline (speedup 1.0000x reference)
import functools
import math

import numpy as np

import jax
import jax.numpy as jnp
from jax import lax
from jax.experimental import pallas as pl
from jax.experimental.pallas import tpu as pltpu
from jax.experimental.pallas import tpu_sc as plsc

_LOG2 = math.log(2.0)

_NC = 2
_NS = 16
_NW = _NC * _NS


def _ssp(x):
    return jnp.maximum(x, 0.0) + jnp.log(1.0 + jnp.exp(-jnp.abs(x))) - _LOG2


def _precompute_body(node_ref, batch_ref, glob_ref, w1s_ref, w1d_ref,
                     w1g_ref, b1_ref, a_ref, b_ref):
    x = node_ref[...]
    gp = jnp.dot(glob_ref[...], w1g_ref[...],
                 preferred_element_type=jnp.float32)
    n = x.shape[0]
    g = gp.shape[0]
    onehot = (lax.broadcasted_iota(jnp.int32, (n, g), 1)
              == batch_ref[...]).astype(jnp.float32)
    a_ref[...] = (jnp.dot(x, w1s_ref[...], preferred_element_type=jnp.float32)
                  + jnp.dot(onehot, gp, preferred_element_type=jnp.float32)
                  + b1_ref[...])
    b_ref[...] = jnp.dot(x, w1d_ref[...], preferred_element_type=jnp.float32)


def _precompute_tables(node_feats, batch2d, global_feats, w1s, w1d, w1g, b1row):
    n = node_feats.shape[0]
    h2 = w1s.shape[1]
    return pl.pallas_call(
        _precompute_body,
        out_shape=(
            jax.ShapeDtypeStruct((n, h2), jnp.float32),
            jax.ShapeDtypeStruct((n, h2), jnp.float32),
        ),
    )(node_feats, batch2d, global_feats, w1s, w1d, w1g, b1row)


def _make_sc_gather(n_edges, h2, chunk, nbuf):
    ew = n_edges // _NW
    nchunk = ew // chunk
    assert nchunk % nbuf == 0

    mesh = plsc.VectorSubcoreMesh(core_axis_name="c", subcore_axis_name="s",
                                  num_cores=_NC, num_subcores=_NS)

    @functools.partial(
        pl.kernel,
        out_type=(
            jax.ShapeDtypeStruct((n_edges, h2), jnp.float32),
            jax.ShapeDtypeStruct((n_edges, h2), jnp.float32),
        ),
        mesh=mesh,
        scratch_types=[
            pltpu.VMEM((ew,), jnp.int32),
            pltpu.VMEM((ew,), jnp.int32),
            pltpu.VMEM((nbuf, chunk, h2), jnp.float32),
            pltpu.VMEM((nbuf, chunk, h2), jnp.float32),
            pltpu.SemaphoreType.DMA((nbuf,)),
            pltpu.SemaphoreType.DMA((nbuf,)),
            pltpu.SemaphoreType.DMA((nbuf,)),
            pltpu.SemaphoreType.DMA((nbuf,)),
        ],
    )
    def gather_rows(a_hbm, b_hbm, src_hbm, dst_hbm, outa_hbm, outb_hbm,
                    idxs_v, idxd_v, rowa_v, rowb_v, sema, semb,
                    semwa, semwb):
        wid = lax.axis_index("s") * _NC + lax.axis_index("c")
        base = wid * ew

        pltpu.sync_copy(src_hbm.at[pl.ds(base, ew)], idxs_v)
        pltpu.sync_copy(dst_hbm.at[pl.ds(base, ew)], idxd_v)

        def fire(c, b):
            pltpu.async_copy(a_hbm.at[idxs_v.at[pl.ds(c * chunk, chunk)]],
                             rowa_v.at[b], sema.at[b])
            pltpu.async_copy(b_hbm.at[idxd_v.at[pl.ds(c * chunk, chunk)]],
                             rowb_v.at[b], semb.at[b])

        def wait_write(c, b):
            pltpu.make_async_copy(
                rowa_v.at[b],
                outa_hbm.at[pl.ds(base + c * chunk, chunk)],
                semwa.at[b]).wait()
            pltpu.make_async_copy(
                rowb_v.at[b],
                outb_hbm.at[pl.ds(base + c * chunk, chunk)],
                semwb.at[b]).wait()

        for b in range(nbuf - 1):
            fire(b, b)

        def group_body(g, carry):
            for j in range(nbuf):
                b = j
                c = g * nbuf + j
                pltpu.make_async_copy(
                    a_hbm.at[idxs_v.at[pl.ds(c * chunk, chunk)]],
                    rowa_v.at[b], sema.at[b]).wait()
                pltpu.make_async_copy(
                    b_hbm.at[idxd_v.at[pl.ds(c * chunk, chunk)]],
                    rowb_v.at[b], semb.at[b]).wait()

                pltpu.async_copy(rowa_v.at[b],
                                 outa_hbm.at[pl.ds(base + c * chunk, chunk)],
                                 semwa.at[b])
                pltpu.async_copy(rowb_v.at[b],
                                 outb_hbm.at[pl.ds(base + c * chunk, chunk)],
                                 semwb.at[b])

                c2 = c + nbuf - 1
                b2 = (j + nbuf - 1) % nbuf

                @pl.when((c2 < nchunk) & (c2 >= nbuf))
                def _():
                    wait_write(c2 - nbuf, b2)

                @pl.when(c2 < nchunk)
                def _():
                    fire(c2, b2)

            return carry

        lax.fori_loop(0, nchunk // nbuf, group_body, 0)

        for k in range(nbuf):
            c = nchunk - nbuf + k
            wait_write(c, c % nbuf)

    return gather_rows


def _mlp_body(ga_ref, gb_ref, ef_ref, w1e_ref, w2_ref, b2_ref, w3_ref,
              b3_ref, out_ref):
    bf = jnp.bfloat16
    x = (ga_ref[...] + gb_ref[...]
         + jnp.dot(ef_ref[...], w1e_ref[...],
                   preferred_element_type=jnp.float32))
    h = _ssp(x)
    h = _ssp(jnp.dot(h.astype(bf), w2_ref[...],
                     preferred_element_type=jnp.float32) + b2_ref[...])
    out_ref[...] = _ssp(jnp.dot(h.astype(bf), w3_ref[...],
                                preferred_element_type=jnp.float32)
                        + b3_ref[...])


def _mlp(ga, gb, edge_feats, w1e, w2, b2row, w3, b3row, block_e):
    e, h2 = ga.shape
    de = edge_feats.shape[1]
    h = w3.shape[1]
    grid = e // block_e
    return pl.pallas_call(
        _mlp_body,
        grid=(grid,),
        in_specs=[
            pl.BlockSpec((block_e, h2), lambda i: (i, 0)),
            pl.BlockSpec((block_e, h2), lambda i: (i, 0)),
            pl.BlockSpec((block_e, de), lambda i: (i, 0)),
            pl.BlockSpec((de, h2), lambda i: (0, 0)),
            pl.BlockSpec((h2, h2), lambda i: (0, 0)),
            pl.BlockSpec((1, h2), lambda i: (0, 0)),
            pl.BlockSpec((h2, h), lambda i: (0, 0)),
            pl.BlockSpec((1, h), lambda i: (0, 0)),
        ],
        out_specs=pl.BlockSpec((block_e, h), lambda i: (i, 0)),
        out_shape=jax.ShapeDtypeStruct((e, h), jnp.float32),
    )(ga, gb, edge_feats, w1e, w2, b2row, w3, b3row)


def kernel(node_feats, edge_feats, global_feats, edge_index, batch,
           W1, b1, W2, b2, W3, b3):
    n, d_node = node_feats.shape
    e, d_edge = edge_feats.shape
    h2 = W1.shape[1]

    w1s = W1[:d_node]
    w1d = W1[d_node:2 * d_node]
    w1e = W1[2 * d_node:2 * d_node + d_edge]
    w1g = W1[2 * d_node + d_edge:]

    idx_src = edge_index[0].astype(jnp.int32)
    idx_dst = edge_index[1].astype(jnp.int32)
    batch2d = batch.astype(jnp.int32).reshape(n, 1)

    a_tab, b_tab = _precompute_tables(node_feats, batch2d, global_feats,
                                      w1s, w1d, w1g, b1.reshape(1, h2))

    chunk = 80
    nbuf = 5
    assert e % (_NW * chunk) == 0
    assert h2 == 128
    ga, gb = _make_sc_gather(e, h2, chunk, nbuf)(a_tab, b_tab,
                                                 idx_src, idx_dst)

    block_e = 4000
    assert e % block_e == 0
    bf = jnp.bfloat16
    return _mlp(ga, gb, edge_feats.astype(bf), w1e.astype(bf),
                W2.astype(bf), b2.reshape(1, h2), W3.astype(bf),
                b3.reshape(1, W3.shape[1]), block_e)


# speedup vs baseline: 1.0371x; 1.0371x over previous
import functools
import math

import numpy as np

import jax
import jax.numpy as jnp
from jax import lax
from jax.experimental import pallas as pl
from jax.experimental.pallas import tpu as pltpu
from jax.experimental.pallas import tpu_sc as plsc

_LOG2 = math.log(2.0)

_NC = 2
_NS = 16
_NW = _NC * _NS


def _ssp(x):
    return jnp.maximum(x, 0.0) + jnp.log(1.0 + jnp.exp(-jnp.abs(x))) - _LOG2


def _precompute_body(node_ref, batch_ref, glob_ref, w1s_ref, w1d_ref,
                     w1g_ref, b1_ref, a_ref, b_ref):
    x = node_ref[...]
    gp = jnp.dot(glob_ref[...], w1g_ref[...],
                 preferred_element_type=jnp.float32)
    n = x.shape[0]
    g = gp.shape[0]
    onehot = (lax.broadcasted_iota(jnp.int32, (n, g), 1)
              == batch_ref[...]).astype(jnp.float32)
    a_ref[...] = (jnp.dot(x, w1s_ref[...], preferred_element_type=jnp.float32)
                  + jnp.dot(onehot, gp, preferred_element_type=jnp.float32)
                  + b1_ref[...])
    b_ref[...] = jnp.dot(x, w1d_ref[...], preferred_element_type=jnp.float32)


def _precompute_tables(node_feats, batch2d, global_feats, w1s, w1d, w1g, b1row):
    n = node_feats.shape[0]
    h2 = w1s.shape[1]
    return pl.pallas_call(
        _precompute_body,
        out_shape=(
            jax.ShapeDtypeStruct((n, h2), jnp.float32),
            jax.ShapeDtypeStruct((n, h2), jnp.float32),
        ),
    )(node_feats, batch2d, global_feats, w1s, w1d, w1g, b1row)


def _make_sc_gather(n_edges, h2, chunk, nbuf):
    ew = n_edges // _NW
    nchunk = ew // chunk
    assert nchunk % nbuf == 0

    mesh = plsc.VectorSubcoreMesh(core_axis_name="c", subcore_axis_name="s",
                                  num_cores=_NC, num_subcores=_NS)

    @functools.partial(
        pl.kernel,
        out_type=(
            jax.ShapeDtypeStruct((n_edges, h2), jnp.float32),
            jax.ShapeDtypeStruct((n_edges, h2), jnp.float32),
        ),
        mesh=mesh,
        scratch_types=[
            pltpu.VMEM((ew,), jnp.int32),
            pltpu.VMEM((ew,), jnp.int32),
            pltpu.VMEM((nbuf, chunk, h2), jnp.float32),
            pltpu.VMEM((nbuf, chunk, h2), jnp.float32),
            pltpu.SemaphoreType.DMA((nbuf,)),
            pltpu.SemaphoreType.DMA((nbuf,)),
            pltpu.SemaphoreType.DMA((nbuf,)),
            pltpu.SemaphoreType.DMA((nbuf,)),
        ],
    )
    def gather_rows(a_hbm, b_hbm, src_hbm, dst_hbm, outa_hbm, outb_hbm,
                    idxs_v, idxd_v, rowa_v, rowb_v, sema, semb,
                    semwa, semwb):
        wid = lax.axis_index("s") * _NC + lax.axis_index("c")
        base = wid * ew

        pltpu.sync_copy(src_hbm.at[pl.ds(base, ew)], idxs_v)
        pltpu.sync_copy(dst_hbm.at[pl.ds(base, ew)], idxd_v)

        def fire(c, b):
            pltpu.async_copy(a_hbm.at[idxs_v.at[pl.ds(c * chunk, chunk)]],
                             rowa_v.at[b], sema.at[b])
            pltpu.async_copy(b_hbm.at[idxd_v.at[pl.ds(c * chunk, chunk)]],
                             rowb_v.at[b], semb.at[b])

        def wait_write(c, b):
            pltpu.make_async_copy(
                rowa_v.at[b],
                outa_hbm.at[pl.ds(base + c * chunk, chunk)],
                semwa.at[b]).wait()
            pltpu.make_async_copy(
                rowb_v.at[b],
                outb_hbm.at[pl.ds(base + c * chunk, chunk)],
                semwb.at[b]).wait()

        for b in range(nbuf - 1):
            fire(b, b)

        def group_body(g, carry):
            for j in range(nbuf):
                b = j
                c = g * nbuf + j
                pltpu.make_async_copy(
                    a_hbm.at[idxs_v.at[pl.ds(c * chunk, chunk)]],
                    rowa_v.at[b], sema.at[b]).wait()
                pltpu.make_async_copy(
                    b_hbm.at[idxd_v.at[pl.ds(c * chunk, chunk)]],
                    rowb_v.at[b], semb.at[b]).wait()

                pltpu.async_copy(rowa_v.at[b],
                                 outa_hbm.at[pl.ds(base + c * chunk, chunk)],
                                 semwa.at[b])
                pltpu.async_copy(rowb_v.at[b],
                                 outb_hbm.at[pl.ds(base + c * chunk, chunk)],
                                 semwb.at[b])

                c2 = c + nbuf - 1
                b2 = (j + nbuf - 1) % nbuf

                @pl.when((c2 < nchunk) & (c2 >= nbuf))
                def _():
                    wait_write(c2 - nbuf, b2)

                @pl.when(c2 < nchunk)
                def _():
                    fire(c2, b2)

            return carry

        lax.fori_loop(0, nchunk // nbuf, group_body, 0)

        for k in range(nbuf):
            c = nchunk - nbuf + k
            wait_write(c, c % nbuf)

    return gather_rows


def _mlp_body(ga_ref, gb_ref, ef_ref, w1e_ref, w2_ref, b2_ref, w3_ref,
              b3_ref, out_ref):
    bf = jnp.bfloat16
    x = (ga_ref[...] + gb_ref[...]
         + jnp.dot(ef_ref[...], w1e_ref[...],
                   preferred_element_type=jnp.float32))
    h = _ssp(x)
    h = _ssp(jnp.dot(h.astype(bf), w2_ref[...],
                     preferred_element_type=jnp.float32) + b2_ref[...])
    out_ref[...] = _ssp(jnp.dot(h.astype(bf), w3_ref[...],
                                preferred_element_type=jnp.float32)
                        + b3_ref[...])


def _mlp(ga, gb, edge_feats, w1e, w2, b2row, w3, b3row, block_e):
    e, h2 = ga.shape
    de = edge_feats.shape[1]
    h = w3.shape[1]
    grid = e // block_e
    return pl.pallas_call(
        _mlp_body,
        grid=(grid,),
        in_specs=[
            pl.BlockSpec((block_e, h2), lambda i: (i, 0)),
            pl.BlockSpec((block_e, h2), lambda i: (i, 0)),
            pl.BlockSpec((block_e, de), lambda i: (i, 0)),
            pl.BlockSpec((de, h2), lambda i: (0, 0)),
            pl.BlockSpec((h2, h2), lambda i: (0, 0)),
            pl.BlockSpec((1, h2), lambda i: (0, 0)),
            pl.BlockSpec((h2, h), lambda i: (0, 0)),
            pl.BlockSpec((1, h), lambda i: (0, 0)),
        ],
        out_specs=pl.BlockSpec((block_e, h), lambda i: (i, 0)),
        out_shape=jax.ShapeDtypeStruct((e, h), jnp.float32),
    )(ga, gb, edge_feats, w1e, w2, b2row, w3, b3row)


def kernel(node_feats, edge_feats, global_feats, edge_index, batch,
           W1, b1, W2, b2, W3, b3):
    n, d_node = node_feats.shape
    e, d_edge = edge_feats.shape
    h2 = W1.shape[1]

    w1s = W1[:d_node]
    w1d = W1[d_node:2 * d_node]
    w1e = W1[2 * d_node:2 * d_node + d_edge]
    w1g = W1[2 * d_node + d_edge:]

    idx_src = edge_index[0].astype(jnp.int32)
    idx_dst = edge_index[1].astype(jnp.int32)
    batch2d = batch.astype(jnp.int32).reshape(n, 1)

    a_tab, b_tab = _precompute_tables(node_feats, batch2d, global_feats,
                                      w1s, w1d, w1g, b1.reshape(1, h2))

    chunk = 80
    nbuf = 5
    assert e % (_NW * chunk) == 0
    assert h2 == 128
    ga, gb = _make_sc_gather(e, h2, chunk, nbuf)(a_tab, b_tab,
                                                 idx_src, idx_dst)

    block_e = 8000
    assert e % block_e == 0
    bf = jnp.bfloat16
    return _mlp(ga, gb, edge_feats.astype(bf), w1e.astype(bf),
                W2.astype(bf), b2.reshape(1, h2), W3.astype(bf),
                b3.reshape(1, W3.shape[1]), block_e)
